# baseline (speedup 1.0000x reference)
import functools
import math

import jax
import jax.numpy as jnp
from jax import lax
from jax.experimental import pallas as pl
from jax.experimental.pallas import tpu as pltpu
from jax.experimental.pallas import tpu_sc as plsc

D_MODEL = 128
SCALE = math.sqrt(float(D_MODEL))

_info = plsc.get_sparse_core_info()
_NC, _NS, _L = _info.num_cores, _info.num_subcores, _info.num_lanes
_NW = _NC * _NS

CHUNK = 64
NBUF = 14


@functools.lru_cache(maxsize=None)
def _make_kernel(n_rows: int, n_cols: int):
    n_idx = n_rows * n_cols
    b_per_w = n_idx // _NW
    assert n_cols % b_per_w == 0 and b_per_w % CHUNK == 0
    w_per_row = n_cols // b_per_w
    n_chunks = b_per_w // CHUNK
    mesh = plsc.VectorSubcoreMesh(core_axis_name="c", subcore_axis_name="s")

    scratch = [pltpu.VMEM((b_per_w,), jnp.int32)]
    scratch += [pltpu.VMEM((CHUNK, D_MODEL), jnp.float32) for _ in range(NBUF)]
    scratch += [pltpu.SemaphoreType.DMA for _ in range(2 * NBUF)]

    @functools.partial(
        pl.kernel,
        mesh=mesh,
        out_type=jax.ShapeDtypeStruct((n_idx, D_MODEL), jnp.float32),
        scratch_types=scratch,
    )
    def emb(x_hbm, table_hbm, out_hbm, idx_v, *bufs_and_sems):
        bufs = bufs_and_sems[:NBUF]
        gsems = bufs_and_sems[NBUF:2 * NBUF]
        ssems = bufs_and_sems[2 * NBUF:]
        wid = lax.axis_index("s") * _NC + lax.axis_index("c")
        base = wid * b_per_w
        row = wid // w_per_row
        col0 = (wid % w_per_row) * b_per_w
        pltpu.sync_copy(x_hbm.at[row, pl.ds(col0, b_per_w)], idx_v)

        gathers = [None] * NBUF
        stores = [None] * NBUF
        for b in range(min(NBUF, n_chunks)):
            gathers[b] = pltpu.async_copy(
                table_hbm.at[idx_v.at[pl.ds(b * CHUNK, CHUNK)]],
                bufs[b], gsems[b])

        for c in range(n_chunks):
            b = c % NBUF
            gathers[b].wait()
            rows_v = bufs[b]

            def row_body(r, carry, rows_v=rows_v):
                for rr in range(2):
                    for j in range(D_MODEL // _L):
                        sl = pl.ds(j * _L, _L)
                        rows_v[2 * r + rr, sl] = rows_v[2 * r + rr, sl] * SCALE
                return carry

            lax.fori_loop(0, CHUNK // 2, row_body, 0)
            stores[b] = pltpu.async_copy(
                rows_v, out_hbm.at[pl.ds(base + c * CHUNK, CHUNK)], ssems[b])
            nc = c + NBUF
            if nc < n_chunks:
                stores[b].wait()
                gathers[b] = pltpu.async_copy(
                    table_hbm.at[idx_v.at[pl.ds(nc * CHUNK, CHUNK)]],
                    bufs[b], gsems[b])

        for c in range(max(0, n_chunks - NBUF), n_chunks):
            stores[c % NBUF].wait()

    return emb


def kernel(x, table):
    orig_shape = x.shape
    out = _make_kernel(x.shape[0], x.shape[1])(x.astype(jnp.int32), table)
    return out.reshape(*orig_shape, D_MODEL)


# speedup vs baseline: 1.0365x; 1.0365x over previous
import functools
import math

import jax
import jax.numpy as jnp
from jax import lax
from jax.experimental import pallas as pl
from jax.experimental.pallas import tpu as pltpu
from jax.experimental.pallas import tpu_sc as plsc

D_MODEL = 128
SCALE = math.sqrt(float(D_MODEL))

_info = plsc.get_sparse_core_info()
_NC, _NS, _L = _info.num_cores, _info.num_subcores, _info.num_lanes
_NW = _NC * _NS

CHUNK = 128
NBUF = 7


@functools.lru_cache(maxsize=None)
def _make_kernel(n_rows: int, n_cols: int):
    n_idx = n_rows * n_cols
    b_per_w = n_idx // _NW
    assert n_cols % b_per_w == 0 and b_per_w % CHUNK == 0
    w_per_row = n_cols // b_per_w
    n_chunks = b_per_w // CHUNK
    mesh = plsc.VectorSubcoreMesh(core_axis_name="c", subcore_axis_name="s")

    scratch = [pltpu.VMEM((b_per_w,), jnp.int32)]
    scratch += [pltpu.VMEM((CHUNK, D_MODEL), jnp.float32) for _ in range(NBUF)]
    scratch += [pltpu.SemaphoreType.DMA for _ in range(2 * NBUF)]

    @functools.partial(
        pl.kernel,
        mesh=mesh,
        out_type=jax.ShapeDtypeStruct((n_idx, D_MODEL), jnp.float32),
        scratch_types=scratch,
    )
    def emb(x_hbm, table_hbm, out_hbm, idx_v, *bufs_and_sems):
        bufs = bufs_and_sems[:NBUF]
        gsems = bufs_and_sems[NBUF:2 * NBUF]
        ssems = bufs_and_sems[2 * NBUF:]
        wid = lax.axis_index("s") * _NC + lax.axis_index("c")
        base = wid * b_per_w
        row = wid // w_per_row
        col0 = (wid % w_per_row) * b_per_w
        pltpu.sync_copy(x_hbm.at[row, pl.ds(col0, b_per_w)], idx_v)

        gathers = [None] * NBUF
        stores = [None] * NBUF
        for b in range(min(NBUF, n_chunks)):
            gathers[b] = pltpu.async_copy(
                table_hbm.at[idx_v.at[pl.ds(b * CHUNK, CHUNK)]],
                bufs[b], gsems[b])

        for c in range(n_chunks):
            b = c % NBUF
            gathers[b].wait()
            rows_v = bufs[b]

            def row_body(r, carry, rows_v=rows_v):
                for rr in range(2):
                    for j in range(D_MODEL // _L):
                        sl = pl.ds(j * _L, _L)
                        rows_v[2 * r + rr, sl] = rows_v[2 * r + rr, sl] * SCALE
                return carry

            lax.fori_loop(0, CHUNK // 2, row_body, 0)
            stores[b] = pltpu.async_copy(
                rows_v, out_hbm.at[pl.ds(base + c * CHUNK, CHUNK)], ssems[b])
            nc = c + NBUF
            if nc < n_chunks:
                stores[b].wait()
                gathers[b] = pltpu.async_copy(
                    table_hbm.at[idx_v.at[pl.ds(nc * CHUNK, CHUNK)]],
                    bufs[b], gsems[b])

        for c in range(max(0, n_chunks - NBUF), n_chunks):
            stores[c % NBUF].wait()

    return emb


def kernel(x, table):
    orig_shape = x.shape
    out = _make_kernel(x.shape[0], x.shape[1])(x.astype(jnp.int32), table)
    return out.reshape(*orig_shape, D_MODEL)
